# baseline (speedup 1.0000x reference)
import functools

import jax
import jax.numpy as jnp
from jax import lax
from jax.experimental import pallas as pl
from jax.experimental.pallas import tpu as pltpu
from jax.experimental.pallas import tpu_sc as plsc

_N = 10000
_NP = 10240
_E = 320000
_D = 128
_H = 128
_C = 16

_CHUNK = 128
_NCHUNKS = _E // _CHUNK
_NWORKERS = 32
_TILES = 16
_RPT = _NP // _TILES
_NCHUNKS_PAD = 2560
_TILE_CHUNKS = _NCHUNKS_PAD // _NWORKERS
_HALF_CHUNKS = _TILE_CHUNKS // 2

_BLK = 1024
_GRID = _NP // _BLK


def _make_edge_agg(width):
    mesh = plsc.VectorSubcoreMesh(core_axis_name="c", subcore_axis_name="s")

    @functools.partial(
        pl.kernel,
        mesh=mesh,
        out_type=jax.ShapeDtypeStruct((2 * _NP, width), jnp.float32),
        scratch_types=[
            pltpu.VMEM_SHARED((_NP, width), jnp.float32),
            pltpu.VMEM((_HALF_CHUNKS, _CHUNK), jnp.int32),
            pltpu.VMEM((_HALF_CHUNKS, _CHUNK), jnp.int32),
            pltpu.VMEM((_CHUNK, width), jnp.float32),
            pltpu.VMEM((_CHUNK, width), jnp.float32),
            pltpu.SemaphoreType.DMA,
            pltpu.SemaphoreType.DMA,
            pltpu.SemaphoreType.DMA,
            pltpu.SemaphoreType.DMA,
        ],
    )
    def agg(y_hbm, src2_hbm, dst2_hbm, zeros_hbm, out_hbm,
            acc, src_v, dst_v, rows0, rows1, sg0, sg1, ss0, ss1):
        c = lax.axis_index("c")
        s = lax.axis_index("s")
        wid = s * 2 + c
        r0 = s * _RPT
        start = _TILE_CHUNKS * wid

        pltpu.sync_copy(zeros_hbm.at[pl.ds(r0, _RPT)], acc.at[pl.ds(r0, _RPT)])
        plsc.subcore_barrier()

        def g_start(buf, sem, i):
            pltpu.async_copy(y_hbm.at[src_v.at[i]], buf, sem)

        def g_wait(buf, sem):
            pltpu.make_async_copy(y_hbm.at[src_v.at[0]], buf, sem).wait()

        def s_start(buf, sem, i):
            pltpu.async_copy(buf, acc.at[dst_v.at[i]], sem, add=True)

        def s_wait(buf, sem):
            pltpu.make_async_copy(buf, acc.at[dst_v.at[0]], sem).wait()

        def pair_body(p, carry):
            i0 = 2 * p
            i1 = 2 * p + 1
            g_wait(rows0, sg0)
            s_start(rows0, ss0, i0)

            @pl.when(p > 0)
            def _():
                s_wait(rows1, ss1)

            g_start(rows1, sg1, i1)
            g_wait(rows1, sg1)
            s_start(rows1, ss1, i1)
            s_wait(rows0, ss0)

            @pl.when(i0 + 2 < _HALF_CHUNKS)
            def _():
                g_start(rows0, sg0, i0 + 2)

            return carry

        for h in range(2):
            pltpu.sync_copy(
                src2_hbm.at[pl.ds(start + h * _HALF_CHUNKS, _HALF_CHUNKS)], src_v)
            pltpu.sync_copy(
                dst2_hbm.at[pl.ds(start + h * _HALF_CHUNKS, _HALF_CHUNKS)], dst_v)
            g_start(rows0, sg0, 0)
            lax.fori_loop(0, _HALF_CHUNKS // 2, pair_body, 0)
            s_wait(rows1, ss1)

        plsc.subcore_barrier()
        pltpu.sync_copy(acc.at[pl.ds(r0, _RPT)],
                        out_hbm.at[pl.ds(c * _NP + r0, _RPT)])

    return agg


_edge_agg_h = _make_edge_agg(_H)


_DEGW = 128

@functools.partial(
    pl.kernel,
    mesh=plsc.VectorSubcoreMesh(core_axis_name="c", subcore_axis_name="s"),
    out_type=jax.ShapeDtypeStruct((2 * _NP, _DEGW), jnp.float32),
    scratch_types=[
        pltpu.VMEM_SHARED((_NP, _DEGW), jnp.float32),
        pltpu.VMEM((_TILE_CHUNKS, _CHUNK), jnp.int32),
        pltpu.VMEM((_CHUNK, _DEGW), jnp.float32),
        pltpu.SemaphoreType.DMA,
    ],
)
def _deg_count(dst2_hbm, zeros_hbm, ones_hbm, out_hbm, acc, dst_v, ones_v, sem):
    c = lax.axis_index("c")
    s = lax.axis_index("s")
    wid = s * 2 + c
    r0 = s * _RPT
    start = _TILE_CHUNKS * wid
    pltpu.sync_copy(dst2_hbm.at[pl.ds(start, _TILE_CHUNKS)], dst_v)
    pltpu.sync_copy(zeros_hbm.at[pl.ds(r0, _RPT)], acc.at[pl.ds(r0, _RPT)])
    pltpu.sync_copy(ones_hbm, ones_v)
    plsc.subcore_barrier()

    def body(i, carry):
        pltpu.sync_copy(ones_v, acc.at[dst_v.at[i]], add=True)
        return carry

    lax.fori_loop(0, _TILE_CHUNKS, body, 0)
    plsc.subcore_barrier()
    pltpu.sync_copy(acc.at[pl.ds(r0, _RPT)],
                    out_hbm.at[pl.ds(c * _NP + r0, _RPT)])


def _dinv_block(d0_ref, d1_ref):
    tot = d0_ref[:, 0:1] + d1_ref[:, 0:1] + 1.0
    return lax.rsqrt(tot)


def _k1_body(d0_ref, d1_ref, x_ref, w_ref, o_ref):
    dinv = _dinv_block(d0_ref, d1_ref)
    o_ref[...] = jnp.dot(x_ref[...], w_ref[...],
                         preferred_element_type=jnp.float32) * dinv


def _k_mid_body(d0_ref, d1_ref, p0_ref, p1_ref, y_ref, b_ref, w_ref, o_ref):
    dinv = _dinv_block(d0_ref, d1_ref)
    h = jax.nn.relu(dinv * (p0_ref[...] + p1_ref[...] + y_ref[...]) + b_ref[...])
    o_ref[...] = jnp.dot(h, w_ref[...],
                         preferred_element_type=jnp.float32) * dinv


def _k_pre_body(d0_ref, d1_ref, p0_ref, p1_ref, y_ref, b_ref, o_ref):
    dinv = _dinv_block(d0_ref, d1_ref)
    h = jax.nn.relu(dinv * (p0_ref[...] + p1_ref[...] + y_ref[...]) + b_ref[...])
    o_ref[...] = h * dinv


def _k_final_body(d0_ref, d1_ref, p0_ref, p1_ref, z_ref, w_ref, b_ref, o_ref):
    dinv = _dinv_block(d0_ref, d1_ref)
    agg = p0_ref[...] + p1_ref[...] + z_ref[...]
    o_ref[...] = dinv * jnp.dot(agg, w_ref[...],
                                preferred_element_type=jnp.float32) + b_ref[...]


def _deg_specs():
    return [
        pl.BlockSpec((_BLK, _DEGW), lambda i: (i, 0)),
        pl.BlockSpec((_BLK, _DEGW), lambda i: (i + _GRID, 0)),
    ]


def _part_specs(width):
    return [
        pl.BlockSpec((_BLK, width), lambda i: (i, 0)),
        pl.BlockSpec((_BLK, width), lambda i: (i + _GRID, 0)),
    ]


def _tc_k1(degp, x, w):
    return pl.pallas_call(
        _k1_body,
        grid=(_GRID,),
        in_specs=_deg_specs() + [
            pl.BlockSpec((_BLK, _D), lambda i: (i, 0)),
            pl.BlockSpec((_D, _H), lambda i: (0, 0)),
        ],
        out_specs=pl.BlockSpec((_BLK, _H), lambda i: (i, 0)),
        out_shape=jax.ShapeDtypeStruct((_NP, _H), jnp.float32),
    )(degp, degp, x, w)


def _tc_k_mid(degp, part, y, b, w, wout):
    return pl.pallas_call(
        _k_mid_body,
        grid=(_GRID,),
        in_specs=_deg_specs() + _part_specs(_H) + [
            pl.BlockSpec((_BLK, _H), lambda i: (i, 0)),
            pl.BlockSpec((1, _H), lambda i: (0, 0)),
            pl.BlockSpec((_H, wout), lambda i: (0, 0)),
        ],
        out_specs=pl.BlockSpec((_BLK, wout), lambda i: (i, 0)),
        out_shape=jax.ShapeDtypeStruct((_NP, wout), jnp.float32),
    )(degp, degp, part, part, y, b, w)


def _tc_k_pre(degp, part, y, b):
    return pl.pallas_call(
        _k_pre_body,
        grid=(_GRID,),
        in_specs=_deg_specs() + _part_specs(_H) + [
            pl.BlockSpec((_BLK, _H), lambda i: (i, 0)),
            pl.BlockSpec((1, _H), lambda i: (0, 0)),
        ],
        out_specs=pl.BlockSpec((_BLK, _H), lambda i: (i, 0)),
        out_shape=jax.ShapeDtypeStruct((_NP, _H), jnp.float32),
    )(degp, degp, part, part, y, b)


def _tc_k_final(degp, part, z, w, b):
    return pl.pallas_call(
        _k_final_body,
        grid=(_GRID,),
        in_specs=_deg_specs() + _part_specs(_H) + [
            pl.BlockSpec((_BLK, _H), lambda i: (i, 0)),
            pl.BlockSpec((_H, _C), lambda i: (0, 0)),
            pl.BlockSpec((1, _C), lambda i: (0, 0)),
        ],
        out_specs=pl.BlockSpec((_BLK, _C), lambda i: (i, 0)),
        out_shape=jax.ShapeDtypeStruct((_NP, _C), jnp.float32),
    )(degp, degp, part, part, z, w, b)


def kernel(x, edge_index, W1, b1, W2, b2, W3, b3):
    idx_pad = jnp.full((_NCHUNKS_PAD - _NCHUNKS, _CHUNK), _N, jnp.int32)
    src2 = jnp.concatenate([edge_index[0].reshape(_NCHUNKS, _CHUNK), idx_pad])
    dst2 = jnp.concatenate([edge_index[1].reshape(_NCHUNKS, _CHUNK), idx_pad])

    x_pad = jnp.concatenate([x, jnp.zeros((_NP - _N, _D), jnp.float32)], axis=0)
    zeros_h = jnp.zeros((_NP, _H), jnp.float32)
    zeros_d = jnp.zeros((_NP, _DEGW), jnp.float32)
    ones_d = jnp.ones((_CHUNK, _DEGW), jnp.float32)

    degp = _deg_count(dst2, zeros_d, ones_d)

    y1 = _tc_k1(degp, x_pad, W1)
    p1 = _edge_agg_h(y1, src2, dst2, zeros_h)
    y2 = _tc_k_mid(degp, p1, y1, b1.reshape(1, _H), W2, _H)
    p2 = _edge_agg_h(y2, src2, dst2, zeros_h)
    z = _tc_k_pre(degp, p2, y2, b2.reshape(1, _H))
    p3 = _edge_agg_h(z, src2, dst2, zeros_h)
    out = _tc_k_final(degp, p3, z, W3, b3.reshape(1, _C))
    return out[:_N]


# speedup vs baseline: 2.8445x; 2.8445x over previous
import functools

import jax
import jax.numpy as jnp
from jax import lax
from jax.experimental import pallas as pl
from jax.experimental.pallas import tpu as pltpu
from jax.experimental.pallas import tpu_sc as plsc

_N = 10000
_NP = 10240
_E = 320000
_D = 128
_H = 128
_C = 16

_CHUNK = 128
_NCHUNKS = _E // _CHUNK
_NWORKERS = 32
_TILES = 16
_RPT = _NP // _TILES
_NCHUNKS_PAD = 2560
_TILE_CHUNKS = _NCHUNKS_PAD // _NWORKERS
_HALF_CHUNKS = _TILE_CHUNKS // 2

_BLK = 1024
_GRID = _NP // _BLK


def _make_edge_agg(width):
    mesh = plsc.VectorSubcoreMesh(core_axis_name="c", subcore_axis_name="s")

    @functools.partial(
        pl.kernel,
        mesh=mesh,
        out_type=jax.ShapeDtypeStruct((2 * _NP, width), jnp.float32),
        scratch_types=[
            pltpu.VMEM_SHARED((_NP, width), jnp.float32),
            pltpu.VMEM((_HALF_CHUNKS, _CHUNK), jnp.int32),
            pltpu.VMEM((_HALF_CHUNKS, _CHUNK), jnp.int32),
            pltpu.VMEM((_CHUNK, width), jnp.float32),
            pltpu.VMEM((_CHUNK, width), jnp.float32),
            pltpu.SemaphoreType.DMA,
            pltpu.SemaphoreType.DMA,
            pltpu.SemaphoreType.DMA,
            pltpu.SemaphoreType.DMA,
        ],
    )
    def agg(y_hbm, src2_hbm, dst2_hbm, zeros_hbm, out_hbm,
            acc, src_v, dst_v, rows0, rows1, sg0, sg1, ss0, ss1):
        c = lax.axis_index("c")
        s = lax.axis_index("s")
        wid = s * 2 + c
        r0 = s * _RPT
        start = _TILE_CHUNKS * wid

        pltpu.sync_copy(zeros_hbm.at[pl.ds(r0, _RPT)], acc.at[pl.ds(r0, _RPT)])
        plsc.subcore_barrier()

        def g_start(buf, sem, i):
            pltpu.async_copy(y_hbm.at[src_v.at[i]], buf, sem)

        def g_wait(buf, sem):
            pltpu.make_async_copy(y_hbm.at[src_v.at[0]], buf, sem).wait()

        def s_start(buf, sem, i):
            pltpu.async_copy(buf, acc.at[dst_v.at[i]], sem, add=True)

        def s_wait(buf, sem):
            pltpu.make_async_copy(buf, acc.at[dst_v.at[0]], sem).wait()

        def pair_body(p, carry):
            i0 = 2 * p
            i1 = 2 * p + 1
            g_wait(rows0, sg0)
            s_start(rows0, ss0, i0)

            @pl.when(p > 0)
            def _():
                s_wait(rows1, ss1)

            g_start(rows1, sg1, i1)
            g_wait(rows1, sg1)
            s_start(rows1, ss1, i1)
            s_wait(rows0, ss0)

            @pl.when(i0 + 2 < _HALF_CHUNKS)
            def _():
                g_start(rows0, sg0, i0 + 2)

            return carry

        for h in range(2):
            pltpu.sync_copy(
                src2_hbm.at[pl.ds(start + h * _HALF_CHUNKS, _HALF_CHUNKS)], src_v)
            pltpu.sync_copy(
                dst2_hbm.at[pl.ds(start + h * _HALF_CHUNKS, _HALF_CHUNKS)], dst_v)
            g_start(rows0, sg0, 0)
            lax.fori_loop(0, _HALF_CHUNKS // 2, pair_body, 0)
            s_wait(rows1, ss1)

        plsc.subcore_barrier()
        pltpu.sync_copy(acc.at[pl.ds(r0, _RPT)],
                        out_hbm.at[pl.ds(c * _NP + r0, _RPT)])

    return agg


_edge_agg_h = _make_edge_agg(_H)


_DEGW = 128

@functools.partial(
    pl.kernel,
    mesh=plsc.VectorSubcoreMesh(core_axis_name="c", subcore_axis_name="s"),
    out_type=jax.ShapeDtypeStruct((2 * _NP, _DEGW), jnp.float32),
    scratch_types=[
        pltpu.VMEM_SHARED((_NP, _DEGW), jnp.float32),
        pltpu.VMEM((_TILE_CHUNKS, _CHUNK), jnp.int32),
        pltpu.VMEM((_CHUNK, _DEGW), jnp.float32),
        pltpu.SemaphoreType.DMA,
    ],
)
def _deg_count(dst2_hbm, zeros_hbm, ones_hbm, out_hbm, acc, dst_v, ones_v, sem):
    c = lax.axis_index("c")
    s = lax.axis_index("s")
    wid = s * 2 + c
    r0 = s * _RPT
    start = _TILE_CHUNKS * wid
    pltpu.sync_copy(dst2_hbm.at[pl.ds(start, _TILE_CHUNKS)], dst_v)
    pltpu.sync_copy(zeros_hbm.at[pl.ds(r0, _RPT)], acc.at[pl.ds(r0, _RPT)])
    pltpu.sync_copy(ones_hbm, ones_v)
    plsc.subcore_barrier()

    def body(i, carry):
        pltpu.sync_copy(ones_v, acc.at[dst_v.at[i]], add=True)
        return carry

    lax.fori_loop(0, _TILE_CHUNKS, body, 0)
    plsc.subcore_barrier()
    pltpu.sync_copy(acc.at[pl.ds(r0, _RPT)],
                    out_hbm.at[pl.ds(c * _NP + r0, _RPT)])


def _dinv_block(d0_ref, d1_ref):
    tot = d0_ref[:, 0:1] + d1_ref[:, 0:1] + 1.0
    return lax.rsqrt(tot)


def _k1_body(d0_ref, d1_ref, x_ref, w_ref, o_ref):
    dinv = _dinv_block(d0_ref, d1_ref)
    o_ref[...] = jnp.dot(x_ref[...], w_ref[...],
                         preferred_element_type=jnp.float32) * dinv


def _k_mid_body(d0_ref, d1_ref, p0_ref, p1_ref, y_ref, b_ref, w_ref, o_ref):
    dinv = _dinv_block(d0_ref, d1_ref)
    h = jax.nn.relu(dinv * (p0_ref[...] + p1_ref[...] + y_ref[...]) + b_ref[...])
    o_ref[...] = jnp.dot(h, w_ref[...],
                         preferred_element_type=jnp.float32) * dinv


def _k_pre_body(d0_ref, d1_ref, p0_ref, p1_ref, y_ref, b_ref, o_ref):
    dinv = _dinv_block(d0_ref, d1_ref)
    h = jax.nn.relu(dinv * (p0_ref[...] + p1_ref[...] + y_ref[...]) + b_ref[...])
    o_ref[...] = h * dinv


def _k_final_body(d0_ref, d1_ref, p0_ref, p1_ref, z_ref, w_ref, b_ref, o_ref):
    dinv = _dinv_block(d0_ref, d1_ref)
    agg = p0_ref[...] + p1_ref[...] + z_ref[...]
    o_ref[...] = dinv * jnp.dot(agg, w_ref[...],
                                preferred_element_type=jnp.float32) + b_ref[...]


def _deg_specs():
    return [
        pl.BlockSpec((_BLK, _DEGW), lambda i: (i, 0)),
        pl.BlockSpec((_BLK, _DEGW), lambda i: (i + _GRID, 0)),
    ]


def _part_specs(width):
    return [
        pl.BlockSpec((_BLK, width), lambda i: (i, 0)),
        pl.BlockSpec((_BLK, width), lambda i: (i + _GRID, 0)),
    ]


def _tc_k1(degp, x, w):
    return pl.pallas_call(
        _k1_body,
        grid=(_GRID,),
        in_specs=_deg_specs() + [
            pl.BlockSpec((_BLK, _D), lambda i: (i, 0)),
            pl.BlockSpec((_D, _H), lambda i: (0, 0)),
        ],
        out_specs=pl.BlockSpec((_BLK, _H), lambda i: (i, 0)),
        out_shape=jax.ShapeDtypeStruct((_NP, _H), jnp.float32),
    )(degp, degp, x, w)


def _tc_k_mid(degp, part, y, b, w, wout):
    return pl.pallas_call(
        _k_mid_body,
        grid=(_GRID,),
        in_specs=_deg_specs() + _part_specs(_H) + [
            pl.BlockSpec((_BLK, _H), lambda i: (i, 0)),
            pl.BlockSpec((1, _H), lambda i: (0, 0)),
            pl.BlockSpec((_H, wout), lambda i: (0, 0)),
        ],
        out_specs=pl.BlockSpec((_BLK, wout), lambda i: (i, 0)),
        out_shape=jax.ShapeDtypeStruct((_NP, wout), jnp.float32),
    )(degp, degp, part, part, y, b, w)


def _tc_k_pre(degp, part, y, b):
    return pl.pallas_call(
        _k_pre_body,
        grid=(_GRID,),
        in_specs=_deg_specs() + _part_specs(_H) + [
            pl.BlockSpec((_BLK, _H), lambda i: (i, 0)),
            pl.BlockSpec((1, _H), lambda i: (0, 0)),
        ],
        out_specs=pl.BlockSpec((_BLK, _H), lambda i: (i, 0)),
        out_shape=jax.ShapeDtypeStruct((_NP, _H), jnp.float32),
    )(degp, degp, part, part, y, b)


def _tc_k_final(degp, part, z, w, b):
    return pl.pallas_call(
        _k_final_body,
        grid=(_GRID,),
        in_specs=_deg_specs() + _part_specs(_H) + [
            pl.BlockSpec((_BLK, _H), lambda i: (i, 0)),
            pl.BlockSpec((_H, _C), lambda i: (0, 0)),
            pl.BlockSpec((1, _C), lambda i: (0, 0)),
        ],
        out_specs=pl.BlockSpec((_BLK, _C), lambda i: (i, 0)),
        out_shape=jax.ShapeDtypeStruct((_NP, _C), jnp.float32),
    )(degp, degp, part, part, z, w, b)


def kernel(x, edge_index, W1, b1, W2, b2, W3, b3):
    n_pad_edges = (_NCHUNKS_PAD - _NCHUNKS) * _CHUNK
    idx_pad = (_N + jnp.arange(n_pad_edges, dtype=jnp.int32) % (_NP - _N)
               ).reshape(_NCHUNKS_PAD - _NCHUNKS, _CHUNK)
    src2 = jnp.concatenate([edge_index[0].reshape(_NCHUNKS, _CHUNK), idx_pad])
    dst2 = jnp.concatenate([edge_index[1].reshape(_NCHUNKS, _CHUNK), idx_pad])

    x_pad = jnp.concatenate([x, jnp.zeros((_NP - _N, _D), jnp.float32)], axis=0)
    zeros_h = jnp.zeros((_NP, _H), jnp.float32)
    zeros_d = jnp.zeros((_NP, _DEGW), jnp.float32)
    ones_d = jnp.ones((_CHUNK, _DEGW), jnp.float32)

    degp = _deg_count(dst2, zeros_d, ones_d)

    y1 = _tc_k1(degp, x_pad, W1)
    p1 = _edge_agg_h(y1, src2, dst2, zeros_h)
    y2 = _tc_k_mid(degp, p1, y1, b1.reshape(1, _H), W2, _H)
    p2 = _edge_agg_h(y2, src2, dst2, zeros_h)
    z = _tc_k_pre(degp, p2, y2, b2.reshape(1, _H))
    p3 = _edge_agg_h(z, src2, dst2, zeros_h)
    out = _tc_k_final(degp, p3, z, W3, b3.reshape(1, _C))
    return out[:_N]


# speedup vs baseline: 2.8615x; 1.0060x over previous
import functools

import jax
import jax.numpy as jnp
from jax import lax
from jax.experimental import pallas as pl
from jax.experimental.pallas import tpu as pltpu
from jax.experimental.pallas import tpu_sc as plsc

_N = 10000
_NP = 10240
_E = 320000
_D = 128
_H = 128
_C = 16

_CHUNK = 128
_NCHUNKS = _E // _CHUNK
_NWORKERS = 32
_TILES = 16
_RPT = _NP // _TILES
_NCHUNKS_PAD = 2560
_TILE_CHUNKS = _NCHUNKS_PAD // _NWORKERS
_HALF_CHUNKS = _TILE_CHUNKS // 2

_BLK = 1024
_GRID = _NP // _BLK


def _make_edge_agg(width):
    mesh = plsc.VectorSubcoreMesh(core_axis_name="c", subcore_axis_name="s")

    @functools.partial(
        pl.kernel,
        mesh=mesh,
        out_type=jax.ShapeDtypeStruct((2 * _NP, width), jnp.float32),
        scratch_types=[
            pltpu.VMEM_SHARED((_NP, width), jnp.float32),
            pltpu.VMEM((_HALF_CHUNKS, _CHUNK), jnp.int32),
            pltpu.VMEM((_HALF_CHUNKS, _CHUNK), jnp.int32),
            pltpu.VMEM((_CHUNK, width), jnp.float32),
            pltpu.VMEM((_CHUNK, width), jnp.float32),
            pltpu.SemaphoreType.DMA,
            pltpu.SemaphoreType.DMA,
            pltpu.SemaphoreType.DMA,
            pltpu.SemaphoreType.DMA,
            pltpu.SemaphoreType.DMA,
        ],
    )
    def agg(y_hbm, src2_hbm, dst2_hbm, zeros_hbm, out_hbm,
            acc, src_v, dst_v, rows0, rows1, sg0, sg1, ss0, ss1, si):
        c = lax.axis_index("c")
        s = lax.axis_index("s")
        wid = s * 2 + c
        r0 = s * _RPT
        start = _TILE_CHUNKS * wid

        zdesc = pltpu.async_copy(zeros_hbm.at[pl.ds(r0, _RPT)],
                                 acc.at[pl.ds(r0, _RPT)], si)

        def g_start(buf, sem, i):
            pltpu.async_copy(y_hbm.at[src_v.at[i]], buf, sem)

        def g_wait(buf, sem):
            pltpu.make_async_copy(y_hbm.at[src_v.at[0]], buf, sem).wait()

        def s_start(buf, sem, i):
            pltpu.async_copy(buf, acc.at[dst_v.at[i]], sem, add=True)

        def s_wait(buf, sem):
            pltpu.make_async_copy(buf, acc.at[dst_v.at[0]], sem).wait()

        def pair_body(p, carry):
            i0 = 2 * p
            i1 = 2 * p + 1
            g_wait(rows0, sg0)
            s_start(rows0, ss0, i0)

            @pl.when(p > 0)
            def _():
                s_wait(rows1, ss1)

            g_start(rows1, sg1, i1)
            g_wait(rows1, sg1)
            s_start(rows1, ss1, i1)
            s_wait(rows0, ss0)

            @pl.when(i0 + 2 < _HALF_CHUNKS)
            def _():
                g_start(rows0, sg0, i0 + 2)

            return carry

        for h in range(2):
            pltpu.sync_copy(
                src2_hbm.at[pl.ds(start + h * _HALF_CHUNKS, _HALF_CHUNKS)], src_v)
            pltpu.sync_copy(
                dst2_hbm.at[pl.ds(start + h * _HALF_CHUNKS, _HALF_CHUNKS)], dst_v)
            if h == 0:
                zdesc.wait()
                plsc.subcore_barrier()
            g_start(rows0, sg0, 0)
            lax.fori_loop(0, _HALF_CHUNKS // 2, pair_body, 0)
            s_wait(rows1, ss1)

        plsc.subcore_barrier()
        pltpu.sync_copy(acc.at[pl.ds(r0, _RPT)],
                        out_hbm.at[pl.ds(c * _NP + r0, _RPT)])

    return agg


_edge_agg_h = _make_edge_agg(_H)


_DEGW = 128
_DEGO = 8

@functools.partial(
    pl.kernel,
    mesh=plsc.VectorSubcoreMesh(core_axis_name="c", subcore_axis_name="s"),
    out_type=jax.ShapeDtypeStruct((2 * _NP, _DEGW), jnp.float32),
    scratch_types=[
        pltpu.VMEM_SHARED((_NP, _DEGW), jnp.float32),
        pltpu.VMEM((_TILE_CHUNKS, _CHUNK), jnp.int32),
        pltpu.VMEM((_CHUNK, _DEGW), jnp.float32),
        pltpu.SemaphoreType.DMA,
    ],
)
def _deg_count(dst2_hbm, zeros_hbm, ones_hbm, out_hbm, acc, dst_v, ones_v, sem):
    c = lax.axis_index("c")
    s = lax.axis_index("s")
    wid = s * 2 + c
    r0 = s * _RPT
    start = _TILE_CHUNKS * wid
    pltpu.sync_copy(dst2_hbm.at[pl.ds(start, _TILE_CHUNKS)], dst_v)
    pltpu.sync_copy(zeros_hbm.at[pl.ds(r0, _RPT)], acc.at[pl.ds(r0, _RPT)])
    pltpu.sync_copy(ones_hbm, ones_v)
    plsc.subcore_barrier()

    def body(i, carry):
        pltpu.sync_copy(ones_v, acc.at[dst_v.at[i]], add=True)
        return carry

    lax.fori_loop(0, _TILE_CHUNKS, body, 0)
    plsc.subcore_barrier()
    pltpu.sync_copy(acc.at[pl.ds(r0, _RPT)],
                    out_hbm.at[pl.ds(c * _NP + r0, _RPT)])


def _dinv_block(d0_ref, d1_ref):
    tot = d0_ref[:, 0:1] + d1_ref[:, 0:1] + 1.0
    return lax.rsqrt(tot)


def _k1_body(d0_ref, d1_ref, x_ref, w_ref, o_ref):
    dinv = _dinv_block(d0_ref, d1_ref)
    o_ref[...] = jnp.dot(x_ref[...], w_ref[...],
                         preferred_element_type=jnp.float32) * dinv


def _k_mid_body(d0_ref, d1_ref, p0_ref, p1_ref, y_ref, b_ref, w_ref, o_ref):
    dinv = _dinv_block(d0_ref, d1_ref)
    h = jax.nn.relu(dinv * (p0_ref[...] + p1_ref[...] + y_ref[...]) + b_ref[...])
    o_ref[...] = jnp.dot(h, w_ref[...],
                         preferred_element_type=jnp.float32) * dinv


def _k_pre_body(d0_ref, d1_ref, p0_ref, p1_ref, y_ref, b_ref, o_ref):
    dinv = _dinv_block(d0_ref, d1_ref)
    h = jax.nn.relu(dinv * (p0_ref[...] + p1_ref[...] + y_ref[...]) + b_ref[...])
    o_ref[...] = h * dinv


def _k_final_body(d0_ref, d1_ref, p0_ref, p1_ref, z_ref, w_ref, b_ref, o_ref):
    dinv = _dinv_block(d0_ref, d1_ref)
    agg = p0_ref[...] + p1_ref[...] + z_ref[...]
    o_ref[...] = dinv * jnp.dot(agg, w_ref[...],
                                preferred_element_type=jnp.float32) + b_ref[...]


def _deg_specs():
    return [
        pl.BlockSpec((_BLK, _DEGW), lambda i: (i, 0)),
        pl.BlockSpec((_BLK, _DEGW), lambda i: (i + _GRID, 0)),
    ]


def _part_specs(width):
    return [
        pl.BlockSpec((_BLK, width), lambda i: (i, 0)),
        pl.BlockSpec((_BLK, width), lambda i: (i + _GRID, 0)),
    ]


def _tc_k1(degp, x, w):
    return pl.pallas_call(
        _k1_body,
        grid=(_GRID,),
        in_specs=_deg_specs() + [
            pl.BlockSpec((_BLK, _D), lambda i: (i, 0)),
            pl.BlockSpec((_D, _H), lambda i: (0, 0)),
        ],
        out_specs=pl.BlockSpec((_BLK, _H), lambda i: (i, 0)),
        out_shape=jax.ShapeDtypeStruct((_NP, _H), jnp.float32),
    )(degp, degp, x, w)


def _tc_k_mid(degp, part, y, b, w, wout):
    return pl.pallas_call(
        _k_mid_body,
        grid=(_GRID,),
        in_specs=_deg_specs() + _part_specs(_H) + [
            pl.BlockSpec((_BLK, _H), lambda i: (i, 0)),
            pl.BlockSpec((1, _H), lambda i: (0, 0)),
            pl.BlockSpec((_H, wout), lambda i: (0, 0)),
        ],
        out_specs=pl.BlockSpec((_BLK, wout), lambda i: (i, 0)),
        out_shape=jax.ShapeDtypeStruct((_NP, wout), jnp.float32),
    )(degp, degp, part, part, y, b, w)


def _tc_k_pre(degp, part, y, b):
    return pl.pallas_call(
        _k_pre_body,
        grid=(_GRID,),
        in_specs=_deg_specs() + _part_specs(_H) + [
            pl.BlockSpec((_BLK, _H), lambda i: (i, 0)),
            pl.BlockSpec((1, _H), lambda i: (0, 0)),
        ],
        out_specs=pl.BlockSpec((_BLK, _H), lambda i: (i, 0)),
        out_shape=jax.ShapeDtypeStruct((_NP, _H), jnp.float32),
    )(degp, degp, part, part, y, b)


def _tc_k_final(degp, part, z, w, b):
    return pl.pallas_call(
        _k_final_body,
        grid=(_GRID,),
        in_specs=_deg_specs() + _part_specs(_H) + [
            pl.BlockSpec((_BLK, _H), lambda i: (i, 0)),
            pl.BlockSpec((_H, _C), lambda i: (0, 0)),
            pl.BlockSpec((1, _C), lambda i: (0, 0)),
        ],
        out_specs=pl.BlockSpec((_BLK, _C), lambda i: (i, 0)),
        out_shape=jax.ShapeDtypeStruct((_NP, _C), jnp.float32),
    )(degp, degp, part, part, z, w, b)


def kernel(x, edge_index, W1, b1, W2, b2, W3, b3):
    n_pad_edges = (_NCHUNKS_PAD - _NCHUNKS) * _CHUNK
    idx_pad = (_N + jnp.arange(n_pad_edges, dtype=jnp.int32) % (_NP - _N)
               ).reshape(_NCHUNKS_PAD - _NCHUNKS, _CHUNK)
    src2 = jnp.concatenate([edge_index[0].reshape(_NCHUNKS, _CHUNK), idx_pad])
    dst2 = jnp.concatenate([edge_index[1].reshape(_NCHUNKS, _CHUNK), idx_pad])

    x_pad = jnp.concatenate([x, jnp.zeros((_NP - _N, _D), jnp.float32)], axis=0)
    zeros_h = jnp.zeros((_NP, _H), jnp.float32)
    zeros_d = jnp.zeros((_NP, _DEGW), jnp.float32)
    ones_d = jnp.ones((_CHUNK, _DEGW), jnp.float32)

    degp = _deg_count(dst2, zeros_d, ones_d)

    y1 = _tc_k1(degp, x_pad, W1)
    p1 = _edge_agg_h(y1, src2, dst2, zeros_h)
    y2 = _tc_k_mid(degp, p1, y1, b1.reshape(1, _H), W2, _H)
    p2 = _edge_agg_h(y2, src2, dst2, zeros_h)
    z = _tc_k_pre(degp, p2, y2, b2.reshape(1, _H))
    p3 = _edge_agg_h(z, src2, dst2, zeros_h)
    out = _tc_k_final(degp, p3, z, W3, b3.reshape(1, _C))
    return out[:_N]


# speedup vs baseline: 2.8631x; 1.0006x over previous
import functools

import jax
import jax.numpy as jnp
from jax import lax
from jax.experimental import pallas as pl
from jax.experimental.pallas import tpu as pltpu
from jax.experimental.pallas import tpu_sc as plsc

_N = 10000
_NP = 10240
_E = 320000
_D = 128
_H = 128
_C = 16

_CHUNK = 128
_NCHUNKS = _E // _CHUNK
_NWORKERS = 32
_TILES = 16
_RPT = _NP // _TILES
_NCHUNKS_PAD = 2560
_TILE_CHUNKS = _NCHUNKS_PAD // _NWORKERS
_HALF_CHUNKS = _TILE_CHUNKS // 2

_BLK = 1024
_GRID = _NP // _BLK


def _make_edge_agg(width):
    mesh = plsc.VectorSubcoreMesh(core_axis_name="c", subcore_axis_name="s")

    @functools.partial(
        pl.kernel,
        mesh=mesh,
        out_type=jax.ShapeDtypeStruct((2 * _NP, width), jnp.float32),
        scratch_types=[
            pltpu.VMEM_SHARED((_NP, width), jnp.float32),
            pltpu.VMEM((_HALF_CHUNKS, _CHUNK), jnp.int32),
            pltpu.VMEM((_HALF_CHUNKS, _CHUNK), jnp.int32),
            pltpu.VMEM((_CHUNK, width), jnp.float32),
            pltpu.VMEM((_CHUNK, width), jnp.float32),
            pltpu.SemaphoreType.DMA,
            pltpu.SemaphoreType.DMA,
            pltpu.SemaphoreType.DMA,
            pltpu.SemaphoreType.DMA,
            pltpu.SemaphoreType.DMA,
        ],
    )
    def agg(y_hbm, src2_hbm, dst2_hbm, zeros_hbm, out_hbm,
            acc, src_v, dst_v, rows0, rows1, sg0, sg1, ss0, ss1, si):
        c = lax.axis_index("c")
        s = lax.axis_index("s")
        wid = s * 2 + c
        r0 = s * _RPT
        start = _TILE_CHUNKS * wid

        zdesc = pltpu.async_copy(zeros_hbm.at[pl.ds(r0, _RPT)],
                                 acc.at[pl.ds(r0, _RPT)], si)

        def g_start(buf, sem, i):
            pltpu.async_copy(y_hbm.at[src_v.at[i]], buf, sem)

        def g_wait(buf, sem):
            pltpu.make_async_copy(y_hbm.at[src_v.at[0]], buf, sem).wait()

        def s_start(buf, sem, i):
            pltpu.async_copy(buf, acc.at[dst_v.at[i]], sem, add=True)

        def s_wait(buf, sem):
            pltpu.make_async_copy(buf, acc.at[dst_v.at[0]], sem).wait()

        def pair_body(p, carry):
            i0 = 2 * p
            i1 = 2 * p + 1
            g_wait(rows0, sg0)
            s_start(rows0, ss0, i0)

            @pl.when(p > 0)
            def _():
                s_wait(rows1, ss1)

            g_start(rows1, sg1, i1)
            g_wait(rows1, sg1)
            s_start(rows1, ss1, i1)
            s_wait(rows0, ss0)

            @pl.when(i0 + 2 < _HALF_CHUNKS)
            def _():
                g_start(rows0, sg0, i0 + 2)

            return carry

        for h in range(2):
            pltpu.sync_copy(
                src2_hbm.at[pl.ds(start + h * _HALF_CHUNKS, _HALF_CHUNKS)], src_v)
            pltpu.sync_copy(
                dst2_hbm.at[pl.ds(start + h * _HALF_CHUNKS, _HALF_CHUNKS)], dst_v)
            if h == 0:
                zdesc.wait()
                plsc.subcore_barrier()
            g_start(rows0, sg0, 0)
            lax.fori_loop(0, _HALF_CHUNKS // 2, pair_body, 0)
            s_wait(rows1, ss1)

        plsc.subcore_barrier()
        pltpu.sync_copy(acc.at[pl.ds(r0, _RPT)],
                        out_hbm.at[pl.ds(c * _NP + r0, _RPT)])

    return agg


_edge_agg_h = _make_edge_agg(_H)


_DEGW = 128
_DEGO = 8

@functools.partial(
    pl.kernel,
    mesh=plsc.VectorSubcoreMesh(core_axis_name="c", subcore_axis_name="s"),
    out_type=jax.ShapeDtypeStruct((2 * _NP, _DEGW), jnp.float32),
    scratch_types=[
        pltpu.VMEM_SHARED((_NP, _DEGW), jnp.float32),
        pltpu.VMEM((_TILE_CHUNKS, _CHUNK), jnp.int32),
        pltpu.VMEM((_CHUNK, _DEGW), jnp.float32),
        pltpu.SemaphoreType.DMA,
        pltpu.SemaphoreType.DMA,
    ],
)
def _deg_count(dst2_hbm, zeros_hbm, ones_hbm, out_hbm, acc, dst_v, ones_v,
               sa, sb):
    c = lax.axis_index("c")
    s = lax.axis_index("s")
    wid = s * 2 + c
    r0 = s * _RPT
    start = _TILE_CHUNKS * wid
    pltpu.sync_copy(dst2_hbm.at[pl.ds(start, _TILE_CHUNKS)], dst_v)
    pltpu.sync_copy(zeros_hbm.at[pl.ds(r0, _RPT)], acc.at[pl.ds(r0, _RPT)])
    pltpu.sync_copy(ones_hbm, ones_v)
    plsc.subcore_barrier()

    def s_issue(i, sem):
        pltpu.async_copy(ones_v, acc.at[dst_v.at[i]], sem, add=True)

    def s_wait(sem):
        pltpu.make_async_copy(ones_v, acc.at[dst_v.at[0]], sem).wait()

    s_issue(0, sa)
    s_issue(1, sb)

    def body(p, carry):
        s_wait(sa)
        s_issue(2 * p + 2, sa)
        s_wait(sb)
        s_issue(2 * p + 3, sb)
        return carry

    lax.fori_loop(0, _TILE_CHUNKS // 2 - 1, body, 0)
    s_wait(sa)
    s_wait(sb)
    plsc.subcore_barrier()
    pltpu.sync_copy(acc.at[pl.ds(r0, _RPT)],
                    out_hbm.at[pl.ds(c * _NP + r0, _RPT)])


def _dinv_block(d0_ref, d1_ref):
    tot = d0_ref[:, 0:1] + d1_ref[:, 0:1] + 1.0
    return lax.rsqrt(tot)


def _k1_body(d0_ref, d1_ref, x_ref, w_ref, o_ref):
    dinv = _dinv_block(d0_ref, d1_ref)
    o_ref[...] = jnp.dot(x_ref[...], w_ref[...],
                         preferred_element_type=jnp.float32) * dinv


def _k_mid_body(d0_ref, d1_ref, p0_ref, p1_ref, y_ref, b_ref, w_ref, o_ref):
    dinv = _dinv_block(d0_ref, d1_ref)
    h = jax.nn.relu(dinv * (p0_ref[...] + p1_ref[...] + y_ref[...]) + b_ref[...])
    o_ref[...] = jnp.dot(h, w_ref[...],
                         preferred_element_type=jnp.float32) * dinv


def _k_pre_body(d0_ref, d1_ref, p0_ref, p1_ref, y_ref, b_ref, o_ref):
    dinv = _dinv_block(d0_ref, d1_ref)
    h = jax.nn.relu(dinv * (p0_ref[...] + p1_ref[...] + y_ref[...]) + b_ref[...])
    o_ref[...] = h * dinv


def _k_final_body(d0_ref, d1_ref, p0_ref, p1_ref, z_ref, w_ref, b_ref, o_ref):
    dinv = _dinv_block(d0_ref, d1_ref)
    agg = p0_ref[...] + p1_ref[...] + z_ref[...]
    o_ref[...] = dinv * jnp.dot(agg, w_ref[...],
                                preferred_element_type=jnp.float32) + b_ref[...]


def _deg_specs():
    return [
        pl.BlockSpec((_BLK, _DEGW), lambda i: (i, 0)),
        pl.BlockSpec((_BLK, _DEGW), lambda i: (i + _GRID, 0)),
    ]


def _part_specs(width):
    return [
        pl.BlockSpec((_BLK, width), lambda i: (i, 0)),
        pl.BlockSpec((_BLK, width), lambda i: (i + _GRID, 0)),
    ]


def _tc_k1(degp, x, w):
    return pl.pallas_call(
        _k1_body,
        grid=(_GRID,),
        in_specs=_deg_specs() + [
            pl.BlockSpec((_BLK, _D), lambda i: (i, 0)),
            pl.BlockSpec((_D, _H), lambda i: (0, 0)),
        ],
        out_specs=pl.BlockSpec((_BLK, _H), lambda i: (i, 0)),
        out_shape=jax.ShapeDtypeStruct((_NP, _H), jnp.float32),
    )(degp, degp, x, w)


def _tc_k_mid(degp, part, y, b, w, wout):
    return pl.pallas_call(
        _k_mid_body,
        grid=(_GRID,),
        in_specs=_deg_specs() + _part_specs(_H) + [
            pl.BlockSpec((_BLK, _H), lambda i: (i, 0)),
            pl.BlockSpec((1, _H), lambda i: (0, 0)),
            pl.BlockSpec((_H, wout), lambda i: (0, 0)),
        ],
        out_specs=pl.BlockSpec((_BLK, wout), lambda i: (i, 0)),
        out_shape=jax.ShapeDtypeStruct((_NP, wout), jnp.float32),
    )(degp, degp, part, part, y, b, w)


def _tc_k_pre(degp, part, y, b):
    return pl.pallas_call(
        _k_pre_body,
        grid=(_GRID,),
        in_specs=_deg_specs() + _part_specs(_H) + [
            pl.BlockSpec((_BLK, _H), lambda i: (i, 0)),
            pl.BlockSpec((1, _H), lambda i: (0, 0)),
        ],
        out_specs=pl.BlockSpec((_BLK, _H), lambda i: (i, 0)),
        out_shape=jax.ShapeDtypeStruct((_NP, _H), jnp.float32),
    )(degp, degp, part, part, y, b)


def _tc_k_final(degp, part, z, w, b):
    return pl.pallas_call(
        _k_final_body,
        grid=(_GRID,),
        in_specs=_deg_specs() + _part_specs(_H) + [
            pl.BlockSpec((_BLK, _H), lambda i: (i, 0)),
            pl.BlockSpec((_H, _C), lambda i: (0, 0)),
            pl.BlockSpec((1, _C), lambda i: (0, 0)),
        ],
        out_specs=pl.BlockSpec((_BLK, _C), lambda i: (i, 0)),
        out_shape=jax.ShapeDtypeStruct((_NP, _C), jnp.float32),
    )(degp, degp, part, part, z, w, b)


def kernel(x, edge_index, W1, b1, W2, b2, W3, b3):
    n_pad_edges = (_NCHUNKS_PAD - _NCHUNKS) * _CHUNK
    idx_pad = (_N + jnp.arange(n_pad_edges, dtype=jnp.int32) % (_NP - _N)
               ).reshape(_NCHUNKS_PAD - _NCHUNKS, _CHUNK)
    src2 = jnp.concatenate([edge_index[0].reshape(_NCHUNKS, _CHUNK), idx_pad])
    dst2 = jnp.concatenate([edge_index[1].reshape(_NCHUNKS, _CHUNK), idx_pad])

    x_pad = jnp.concatenate([x, jnp.zeros((_NP - _N, _D), jnp.float32)], axis=0)
    zeros_h = jnp.zeros((_NP, _H), jnp.float32)
    zeros_d = jnp.zeros((_NP, _DEGW), jnp.float32)
    ones_d = jnp.ones((_CHUNK, _DEGW), jnp.float32)

    degp = _deg_count(dst2, zeros_d, ones_d)

    y1 = _tc_k1(degp, x_pad, W1)
    p1 = _edge_agg_h(y1, src2, dst2, zeros_h)
    y2 = _tc_k_mid(degp, p1, y1, b1.reshape(1, _H), W2, _H)
    p2 = _edge_agg_h(y2, src2, dst2, zeros_h)
    z = _tc_k_pre(degp, p2, y2, b2.reshape(1, _H))
    p3 = _edge_agg_h(z, src2, dst2, zeros_h)
    out = _tc_k_final(degp, p3, z, W3, b3.reshape(1, _C))
    return out[:_N]


# speedup vs baseline: 2.8863x; 1.0081x over previous
import functools

import jax
import jax.numpy as jnp
from jax import lax
from jax.experimental import pallas as pl
from jax.experimental.pallas import tpu as pltpu
from jax.experimental.pallas import tpu_sc as plsc

_N = 10000
_NP = 10240
_E = 320000
_D = 128
_H = 128
_C = 16

_CHUNK = 128
_NCHUNKS = _E // _CHUNK
_NWORKERS = 32
_TILES = 16
_RPT = _NP // _TILES
_NCHUNKS_PAD = 2560
_TILE_CHUNKS = _NCHUNKS_PAD // _NWORKERS
_HALF_CHUNKS = _TILE_CHUNKS // 2

_BLK = 1024
_GRID = _NP // _BLK


def _make_edge_agg(width):
    mesh = plsc.VectorSubcoreMesh(core_axis_name="c", subcore_axis_name="s")

    @functools.partial(
        pl.kernel,
        mesh=mesh,
        out_type=jax.ShapeDtypeStruct((2 * _NP, width), jnp.float32),
        scratch_types=[
            pltpu.VMEM_SHARED((_NP, width), jnp.float32),
            pltpu.VMEM((_HALF_CHUNKS, _CHUNK), jnp.int32),
            pltpu.VMEM((_HALF_CHUNKS, _CHUNK), jnp.int32),
            pltpu.VMEM((_CHUNK, width), jnp.float32),
            pltpu.VMEM((_CHUNK, width), jnp.float32),
            pltpu.SemaphoreType.DMA,
            pltpu.SemaphoreType.DMA,
            pltpu.SemaphoreType.DMA,
            pltpu.SemaphoreType.DMA,
            pltpu.SemaphoreType.DMA,
        ],
    )
    def agg(y_hbm, src2_hbm, dst2_hbm, zeros_hbm, out_hbm,
            acc, src_v, dst_v, rows0, rows1, sg0, sg1, ss0, ss1, si):
        c = lax.axis_index("c")
        s = lax.axis_index("s")
        wid = s * 2 + c
        r0 = s * _RPT
        start = _TILE_CHUNKS * wid

        zdesc = pltpu.async_copy(zeros_hbm.at[pl.ds(r0, _RPT)],
                                 acc.at[pl.ds(r0, _RPT)], si)

        def g_start(buf, sem, i):
            pltpu.async_copy(y_hbm.at[src_v.at[i]], buf, sem)

        def g_wait(buf, sem):
            pltpu.make_async_copy(y_hbm.at[src_v.at[0]], buf, sem).wait()

        def s_start(buf, sem, i):
            pltpu.async_copy(buf, acc.at[dst_v.at[i]], sem, add=True)

        def s_wait(buf, sem):
            pltpu.make_async_copy(buf, acc.at[dst_v.at[0]], sem).wait()

        def pair_body(p, carry):
            i0 = 2 * p
            i1 = 2 * p + 1
            g_wait(rows0, sg0)
            s_start(rows0, ss0, i0)

            @pl.when(p > 0)
            def _():
                s_wait(rows1, ss1)

            g_start(rows1, sg1, i1)
            g_wait(rows1, sg1)
            s_start(rows1, ss1, i1)
            s_wait(rows0, ss0)

            @pl.when(i0 + 2 < _HALF_CHUNKS)
            def _():
                g_start(rows0, sg0, i0 + 2)

            return carry

        for h in range(2):
            pltpu.sync_copy(
                src2_hbm.at[pl.ds(start + h * _HALF_CHUNKS, _HALF_CHUNKS)], src_v)
            pltpu.sync_copy(
                dst2_hbm.at[pl.ds(start + h * _HALF_CHUNKS, _HALF_CHUNKS)], dst_v)
            if h == 0:
                zdesc.wait()
                plsc.subcore_barrier()
            g_start(rows0, sg0, 0)
            lax.fori_loop(0, _HALF_CHUNKS // 2, pair_body, 0)
            s_wait(rows1, ss1)

        plsc.subcore_barrier()
        pltpu.sync_copy(acc.at[pl.ds(r0, _RPT)],
                        out_hbm.at[pl.ds(c * _NP + r0, _RPT)])

    return agg


_edge_agg_h = _make_edge_agg(_H)


_DEGW = 128
_DEGO = 8

@functools.partial(
    pl.kernel,
    mesh=plsc.VectorSubcoreMesh(core_axis_name="c", subcore_axis_name="s"),
    out_type=jax.ShapeDtypeStruct((2 * _NP, _DEGW), jnp.float32),
    scratch_types=[
        pltpu.VMEM_SHARED((_NP, _DEGW), jnp.float32),
        pltpu.VMEM((_TILE_CHUNKS, _CHUNK), jnp.int32),
        pltpu.VMEM((_CHUNK, _DEGW), jnp.float32),
        pltpu.SemaphoreType.DMA,
        pltpu.SemaphoreType.DMA,
    ],
)
def _deg_count(dst2_hbm, zeros_hbm, ones_hbm, out_hbm, acc, dst_v, ones_v,
               sa, sb):
    c = lax.axis_index("c")
    s = lax.axis_index("s")
    wid = s * 2 + c
    r0 = s * _RPT
    start = _TILE_CHUNKS * wid
    pltpu.sync_copy(dst2_hbm.at[pl.ds(start, _TILE_CHUNKS)], dst_v)
    pltpu.sync_copy(zeros_hbm.at[pl.ds(r0, _RPT)], acc.at[pl.ds(r0, _RPT)])
    pltpu.sync_copy(ones_hbm, ones_v)
    plsc.subcore_barrier()

    def s_issue(i, sem):
        pltpu.async_copy(ones_v, acc.at[dst_v.at[i]], sem, add=True)

    def s_wait(sem):
        pltpu.make_async_copy(ones_v, acc.at[dst_v.at[0]], sem).wait()

    s_issue(0, sa)
    s_issue(1, sb)

    def body(p, carry):
        s_wait(sa)
        s_issue(2 * p + 2, sa)
        s_wait(sb)
        s_issue(2 * p + 3, sb)
        return carry

    lax.fori_loop(0, _TILE_CHUNKS // 2 - 1, body, 0)
    s_wait(sa)
    s_wait(sb)
    plsc.subcore_barrier()
    pltpu.sync_copy(acc.at[pl.ds(r0, _RPT)],
                    out_hbm.at[pl.ds(c * _NP + r0, _RPT)])


_DVW = 8


def _k1_body(d0_ref, d1_ref, x_ref, w_ref, o_ref, dv_ref):
    dinv = lax.rsqrt(d0_ref[:, 0:1] + d1_ref[:, 0:1] + 1.0)
    o_ref[...] = jnp.dot(x_ref[...], w_ref[...],
                         preferred_element_type=jnp.float32) * dinv
    dv_ref[...] = jnp.broadcast_to(dinv, (_BLK, _DVW))


def _k_mid_body(dv_ref, p0_ref, p1_ref, y_ref, b_ref, w_ref, o_ref):
    dinv = dv_ref[:, 0:1]
    h = jax.nn.relu(dinv * (p0_ref[...] + p1_ref[...] + y_ref[...]) + b_ref[...])
    o_ref[...] = jnp.dot(h, w_ref[...],
                         preferred_element_type=jnp.float32) * dinv


def _k_pre_body(dv_ref, p0_ref, p1_ref, y_ref, b_ref, o_ref):
    dinv = dv_ref[:, 0:1]
    h = jax.nn.relu(dinv * (p0_ref[...] + p1_ref[...] + y_ref[...]) + b_ref[...])
    o_ref[...] = h * dinv


def _k_final_body(dv_ref, p0_ref, p1_ref, z_ref, w_ref, b_ref, o_ref):
    dinv = dv_ref[:, 0:1]
    agg = p0_ref[...] + p1_ref[...] + z_ref[...]
    o_ref[...] = dinv * jnp.dot(agg, w_ref[...],
                                preferred_element_type=jnp.float32) + b_ref[...]


def _deg_specs():
    return [
        pl.BlockSpec((_BLK, _DEGW), lambda i: (i, 0)),
        pl.BlockSpec((_BLK, _DEGW), lambda i: (i + _GRID, 0)),
    ]


def _dv_spec():
    return [pl.BlockSpec((_BLK, _DVW), lambda i: (i, 0))]


def _part_specs(width):
    return [
        pl.BlockSpec((_BLK, width), lambda i: (i, 0)),
        pl.BlockSpec((_BLK, width), lambda i: (i + _GRID, 0)),
    ]


def _tc_k1(degp, x, w):
    return pl.pallas_call(
        _k1_body,
        grid=(_GRID,),
        in_specs=_deg_specs() + [
            pl.BlockSpec((_BLK, _D), lambda i: (i, 0)),
            pl.BlockSpec((_D, _H), lambda i: (0, 0)),
        ],
        out_specs=[pl.BlockSpec((_BLK, _H), lambda i: (i, 0)),
                   pl.BlockSpec((_BLK, _DVW), lambda i: (i, 0))],
        out_shape=[jax.ShapeDtypeStruct((_NP, _H), jnp.float32),
                   jax.ShapeDtypeStruct((_NP, _DVW), jnp.float32)],
    )(degp, degp, x, w)


def _tc_k_mid(dv, part, y, b, w, wout):
    return pl.pallas_call(
        _k_mid_body,
        grid=(_GRID,),
        in_specs=_dv_spec() + _part_specs(_H) + [
            pl.BlockSpec((_BLK, _H), lambda i: (i, 0)),
            pl.BlockSpec((1, _H), lambda i: (0, 0)),
            pl.BlockSpec((_H, wout), lambda i: (0, 0)),
        ],
        out_specs=pl.BlockSpec((_BLK, wout), lambda i: (i, 0)),
        out_shape=jax.ShapeDtypeStruct((_NP, wout), jnp.float32),
    )(dv, part, part, y, b, w)


def _tc_k_pre(dv, part, y, b):
    return pl.pallas_call(
        _k_pre_body,
        grid=(_GRID,),
        in_specs=_dv_spec() + _part_specs(_H) + [
            pl.BlockSpec((_BLK, _H), lambda i: (i, 0)),
            pl.BlockSpec((1, _H), lambda i: (0, 0)),
        ],
        out_specs=pl.BlockSpec((_BLK, _H), lambda i: (i, 0)),
        out_shape=jax.ShapeDtypeStruct((_NP, _H), jnp.float32),
    )(dv, part, part, y, b)


def _tc_k_final(dv, part, z, w, b):
    return pl.pallas_call(
        _k_final_body,
        grid=(_GRID,),
        in_specs=_dv_spec() + _part_specs(_H) + [
            pl.BlockSpec((_BLK, _H), lambda i: (i, 0)),
            pl.BlockSpec((_H, _C), lambda i: (0, 0)),
            pl.BlockSpec((1, _C), lambda i: (0, 0)),
        ],
        out_specs=pl.BlockSpec((_BLK, _C), lambda i: (i, 0)),
        out_shape=jax.ShapeDtypeStruct((_NP, _C), jnp.float32),
    )(dv, part, part, z, w, b)


def kernel(x, edge_index, W1, b1, W2, b2, W3, b3):
    n_pad_edges = (_NCHUNKS_PAD - _NCHUNKS) * _CHUNK
    idx_pad = (_N + jnp.arange(n_pad_edges, dtype=jnp.int32) % (_NP - _N)
               ).reshape(_NCHUNKS_PAD - _NCHUNKS, _CHUNK)
    src2 = jnp.concatenate([edge_index[0].reshape(_NCHUNKS, _CHUNK), idx_pad])
    dst2 = jnp.concatenate([edge_index[1].reshape(_NCHUNKS, _CHUNK), idx_pad])

    x_pad = jnp.concatenate([x, jnp.zeros((_NP - _N, _D), jnp.float32)], axis=0)
    zeros_h = jnp.zeros((_NP, _H), jnp.float32)
    zeros_d = jnp.zeros((_NP, _DEGW), jnp.float32)
    ones_d = jnp.ones((_CHUNK, _DEGW), jnp.float32)

    degp = _deg_count(dst2, zeros_d, ones_d)

    y1, dv = _tc_k1(degp, x_pad, W1)
    p1 = _edge_agg_h(y1, src2, dst2, zeros_h)
    y2 = _tc_k_mid(dv, p1, y1, b1.reshape(1, _H), W2, _H)
    p2 = _edge_agg_h(y2, src2, dst2, zeros_h)
    z = _tc_k_pre(dv, p2, y2, b2.reshape(1, _H))
    p3 = _edge_agg_h(z, src2, dst2, zeros_h)
    out = _tc_k_final(dv, p3, z, W3, b3.reshape(1, _C))
    return out[:_N]
